# baseline (speedup 1.0000x reference)
import functools

import jax
import jax.numpy as jnp
from jax import lax
from jax.experimental import pallas as pl
from jax.experimental.pallas import tpu as pltpu
from jax.experimental.pallas import tpu_sc as plsc

N_NODES = 100000
N_ELEM = 128
ROWS = 80
CHUNK = ROWS * N_ELEM
N_CHUNKS = N_NODES // ROWS
NW = 32
N_BIG = N_CHUNKS - (N_CHUNKS // NW) * NW
N_ITERS = N_CHUNKS // NW + 1
NBUF = 8
GPC = ROWS // 16


def _onehot_body(types_hbm, out_hbm, buf0, buf1, buf2, buf3, buf4, buf5,
                 buf6, buf7, types_v, sem_t, sem0, sem1, sem2, sem3, sem4,
                 sem5, sem6, sem7):
    wid = lax.axis_index("s") * 2 + lax.axis_index("c")
    bufs = (buf0, buf1, buf2, buf3, buf4, buf5, buf6, buf7)
    sems = (sem0, sem1, sem2, sem3, sem4, sem5, sem6, sem7)
    iota = lax.iota(jnp.int32, 16)
    ones = jnp.ones((16,), jnp.float32)
    zeros = jnp.zeros((16,), jnp.float32)

    big = wid < N_BIG
    start = jnp.where(big, N_ITERS * wid,
                      (N_ITERS - 1) * wid + N_BIG)
    count = jnp.where(big, N_ITERS, N_ITERS - 1)

    @pl.when(big)
    def _():
        pltpu.make_async_copy(
            types_hbm.at[pl.ds(start * ROWS, N_ITERS * ROWS)],
            types_v, sem_t,
        ).start()

    @pl.when(jnp.logical_not(big))
    def _():
        pltpu.make_async_copy(
            types_hbm.at[pl.ds(start * ROWS, (N_ITERS - 1) * ROWS)],
            types_v.at[pl.ds(0, (N_ITERS - 1) * ROWS)], sem_t,
        ).start()

    def zero_buf(buf):
        @pl.loop(0, CHUNK // 128)
        def _(j):
            base = j * 128
            for k in range(8):
                buf[pl.ds(base + 16 * k, 16)] = zeros

    def scatter(buf, i, vals):
        for j in range(GPC):
            tv = types_v[pl.ds(i * ROWS + 16 * j, 16)]
            idx = (16 * j + iota) * N_ELEM + tv
            plsc.store_scatter(buf, [idx], vals)

    def start_out(buf, sem, c):
        pltpu.make_async_copy(
            buf, out_hbm.at[pl.ds(c * CHUNK, CHUNK)], sem
        ).start()

    def wait_out(buf, sem):
        pltpu.make_async_copy(buf, out_hbm.at[pl.ds(0, CHUNK)], sem).wait()

    for b in range(NBUF):
        zero_buf(bufs[b])
        if b == 0:
            @pl.when(big)
            def _():
                pltpu.make_async_copy(
                    types_hbm.at[pl.ds(0, N_ITERS * ROWS)],
                    types_v, sem_t,
                ).wait()

            @pl.when(jnp.logical_not(big))
            def _():
                pltpu.make_async_copy(
                    types_hbm.at[pl.ds(0, (N_ITERS - 1) * ROWS)],
                    types_v.at[pl.ds(0, (N_ITERS - 1) * ROWS)], sem_t,
                ).wait()
        scatter(bufs[b], b, ones)
        start_out(bufs[b], sems[b], start + b)

    @pl.loop(NBUF, N_ITERS, step=NBUF)
    def _(i0):
        for b in range(NBUF):
            i = i0 + b
            wait_out(bufs[b], sems[b])
            scatter(bufs[b], i - NBUF, zeros)

            @pl.when(i < count)
            def _():
                scatter(bufs[b], i, ones)
                start_out(bufs[b], sems[b], start + i)

    for b in range(NBUF):
        @pl.when(N_ITERS - NBUF + b < count)
        def _():
            wait_out(bufs[b], sems[b])


@jax.jit
def _onehot_sc(atomic_types):
    mesh = plsc.VectorSubcoreMesh(core_axis_name="c", subcore_axis_name="s")
    f = functools.partial(
        pl.kernel,
        mesh=mesh,
        compiler_params=pltpu.CompilerParams(
            needs_layout_passes=False,
            use_tc_tiling_on_sc=False,
        ),
        out_type=jax.ShapeDtypeStruct((N_NODES * N_ELEM,), jnp.float32),
        scratch_types=[
            pltpu.VMEM((CHUNK,), jnp.float32),
            pltpu.VMEM((CHUNK,), jnp.float32),
            pltpu.VMEM((CHUNK,), jnp.float32),
            pltpu.VMEM((CHUNK,), jnp.float32),
            pltpu.VMEM((CHUNK,), jnp.float32),
            pltpu.VMEM((CHUNK,), jnp.float32),
            pltpu.VMEM((CHUNK,), jnp.float32),
            pltpu.VMEM((CHUNK,), jnp.float32),
            pltpu.VMEM((N_ITERS * ROWS,), jnp.int32),
            pltpu.SemaphoreType.DMA,
            pltpu.SemaphoreType.DMA,
            pltpu.SemaphoreType.DMA,
            pltpu.SemaphoreType.DMA,
            pltpu.SemaphoreType.DMA,
            pltpu.SemaphoreType.DMA,
            pltpu.SemaphoreType.DMA,
            pltpu.SemaphoreType.DMA,
            pltpu.SemaphoreType.DMA,
        ],
    )(_onehot_body)
    return f(atomic_types)


def kernel(atomic_types, positions):
    del positions
    return _onehot_sc(atomic_types).reshape(N_NODES, N_ELEM)


# speedup vs baseline: 1.0191x; 1.0191x over previous
import functools

import jax
import jax.numpy as jnp
from jax import lax
from jax.experimental import pallas as pl
from jax.experimental.pallas import tpu as pltpu
from jax.experimental.pallas import tpu_sc as plsc

N_NODES = 100000
N_ELEM = 128
ROWS = 80
CHUNK = ROWS * N_ELEM
N_CHUNKS = N_NODES // ROWS
NW = 32
N_BIG = N_CHUNKS - (N_CHUNKS // NW) * NW
N_ITERS = N_CHUNKS // NW + 1
NBUF = 4
GPC = ROWS // 16


def _onehot_body(types_hbm, out_hbm, buf0, buf1, buf2, buf3, types_v,
                 sem_t, sem0, sem1, sem2, sem3):
    wid = lax.axis_index("s") * 2 + lax.axis_index("c")
    bufs = (buf0, buf1, buf2, buf3)
    sems = (sem0, sem1, sem2, sem3)
    iota = lax.iota(jnp.int32, 16)
    ones = jnp.ones((16,), jnp.float32)
    zeros = jnp.zeros((16,), jnp.float32)

    big = wid < N_BIG
    start = jnp.where(big, N_ITERS * wid,
                      (N_ITERS - 1) * wid + N_BIG)
    count = jnp.where(big, N_ITERS, N_ITERS - 1)

    @pl.when(big)
    def _():
        pltpu.make_async_copy(
            types_hbm.at[pl.ds(start * ROWS, N_ITERS * ROWS)],
            types_v, sem_t,
        ).start()

    @pl.when(jnp.logical_not(big))
    def _():
        pltpu.make_async_copy(
            types_hbm.at[pl.ds(start * ROWS, (N_ITERS - 1) * ROWS)],
            types_v.at[pl.ds(0, (N_ITERS - 1) * ROWS)], sem_t,
        ).start()

    def zero_buf(buf):
        @pl.loop(0, CHUNK // 128)
        def _(j):
            base = j * 128
            for k in range(8):
                buf[pl.ds(base + 16 * k, 16)] = zeros

    def scatter(buf, i, vals):
        for j in range(GPC):
            tv = types_v[pl.ds(i * ROWS + 16 * j, 16)]
            idx = (16 * j + iota) * N_ELEM + tv
            plsc.store_scatter(buf, [idx], vals)

    def start_out(buf, sem, c):
        pltpu.make_async_copy(
            buf, out_hbm.at[pl.ds(c * CHUNK, CHUNK)], sem
        ).start()

    def wait_out(buf, sem):
        pltpu.make_async_copy(buf, out_hbm.at[pl.ds(0, CHUNK)], sem).wait()

    for b in range(NBUF):
        zero_buf(bufs[b])
        if b == 0:
            @pl.when(big)
            def _():
                pltpu.make_async_copy(
                    types_hbm.at[pl.ds(0, N_ITERS * ROWS)],
                    types_v, sem_t,
                ).wait()

            @pl.when(jnp.logical_not(big))
            def _():
                pltpu.make_async_copy(
                    types_hbm.at[pl.ds(0, (N_ITERS - 1) * ROWS)],
                    types_v.at[pl.ds(0, (N_ITERS - 1) * ROWS)], sem_t,
                ).wait()
        scatter(bufs[b], b, ones)
        start_out(bufs[b], sems[b], start + b)

    @pl.loop(NBUF, N_ITERS, step=NBUF)
    def _(i0):
        for b in range(NBUF):
            i = i0 + b
            wait_out(bufs[b], sems[b])
            scatter(bufs[b], i - NBUF, zeros)

            @pl.when(i < count)
            def _():
                scatter(bufs[b], i, ones)
                start_out(bufs[b], sems[b], start + i)

    for b in range(NBUF):
        @pl.when(N_ITERS - NBUF + b < count)
        def _():
            wait_out(bufs[b], sems[b])


@jax.jit
def _onehot_sc(atomic_types):
    mesh = plsc.VectorSubcoreMesh(core_axis_name="c", subcore_axis_name="s")
    f = functools.partial(
        pl.kernel,
        mesh=mesh,
        compiler_params=pltpu.CompilerParams(
            needs_layout_passes=False,
            use_tc_tiling_on_sc=False,
            disable_bounds_checks=True,
            disable_semaphore_checks=True,
        ),
        out_type=jax.ShapeDtypeStruct((N_NODES * N_ELEM,), jnp.float32),
        scratch_types=[
            pltpu.VMEM((CHUNK,), jnp.float32),
            pltpu.VMEM((CHUNK,), jnp.float32),
            pltpu.VMEM((CHUNK,), jnp.float32),
            pltpu.VMEM((CHUNK,), jnp.float32),
            pltpu.VMEM((N_ITERS * ROWS,), jnp.int32),
            pltpu.SemaphoreType.DMA,
            pltpu.SemaphoreType.DMA,
            pltpu.SemaphoreType.DMA,
            pltpu.SemaphoreType.DMA,
            pltpu.SemaphoreType.DMA,
        ],
    )(_onehot_body)
    return f(atomic_types)


def kernel(atomic_types, positions):
    del positions
    return _onehot_sc(atomic_types).reshape(N_NODES, N_ELEM)
